# baseline (speedup 1.0000x reference)
import functools
import math

import jax
import jax.numpy as jnp
from jax import lax
from jax.experimental import pallas as pl
from jax.experimental.pallas import tpu as pltpu
from jax.experimental.pallas import tpu_sc as plsc

_DT_SCALE = float(math.log1p(168.0))
_NUM_CORES = 2
_NUM_SUBCORES = 16
_NW = _NUM_CORES * _NUM_SUBCORES
_LANES = 16
_TBL = 15

_TC_COLS = 512
_TC_ROWS = 64


@functools.lru_cache(maxsize=None)
def _build_sc_kernel(n, s, chunk, unroll):
    per_worker = s // _NW
    n_chunks = per_worker // chunk
    mesh = plsc.VectorSubcoreMesh(
        core_axis_name="c", subcore_axis_name="s",
        num_cores=_NUM_CORES, num_subcores=_NUM_SUBCORES)

    @functools.partial(
        pl.kernel,
        out_type=jax.ShapeDtypeStruct((n,), jnp.float32),
        mesh=mesh,
        compiler_params=pltpu.CompilerParams(
            needs_layout_passes=False, skip_device_barrier=True),
        scratch_types=[
            pltpu.VMEM((_LANES,), jnp.float32),
            pltpu.VMEM((_LANES,), jnp.float32),
            [pltpu.VMEM((chunk,), jnp.float32)] * 2,
            [pltpu.VMEM((chunk,), jnp.float32)] * 2,
            [pltpu.VMEM((chunk,), jnp.int32)] * 2,
            [pltpu.VMEM((chunk,), jnp.float32)] * 2,
            [pltpu.SemaphoreType.DMA] * 2,
            [pltpu.SemaphoreType.DMA] * 2,
        ],
    )
    def sc_kernel(mpc_hbm, dt_hbm, idx_hbm, ltau_hbm, out_hbm,
                  ltau_v, tbl_v, mpc_v, dt_v, idx_v, out_v, in_sem, out_sem):
        wid = lax.axis_index("s") * _NUM_CORES + lax.axis_index("c")
        pltpu.sync_copy(ltau_hbm, ltau_v.at[pl.ds(0, _TBL)])
        tbl_v[...] = -jnp.exp(-ltau_v[...])
        base = wid * per_worker

        def start_in(c, b):
            off = base + c * chunk
            return (
                pltpu.async_copy(mpc_hbm.at[pl.ds(off, chunk)], mpc_v[b], in_sem[b]),
                pltpu.async_copy(dt_hbm.at[pl.ds(off, chunk)], dt_v[b], in_sem[b]),
                pltpu.async_copy(idx_hbm.at[pl.ds(off, chunk)], idx_v[b], in_sem[b]),
            )

        def compute(b):
            @plsc.parallel_loop(0, chunk, step=_LANES, unroll=unroll)
            def _(i):
                sl = pl.ds(pl.multiple_of(i, _LANES), _LANES)
                g = plsc.load_gather(tbl_v, [idx_v[b][sl]])
                e = jnp.exp(dt_v[b][sl] * _DT_SCALE)
                decay = jnp.exp((e - 1.0) * g)
                out_v[b][sl] = 1.0 - (1.0 - mpc_v[b][sl]) * decay

        in_flight = [None, None]
        out_flight = [None, None]
        in_flight[0] = start_in(0, 0)
        for c in range(n_chunks):
            b = c % 2
            if c + 1 < n_chunks:
                in_flight[1 - b] = start_in(c + 1, 1 - b)
            for d in in_flight[b]:
                d.wait()
            if out_flight[b] is not None:
                out_flight[b].wait()
            compute(b)
            off = base + c * chunk
            out_flight[b] = pltpu.async_copy(
                out_v[b], out_hbm.at[pl.ds(off, chunk)], out_sem[b])
        for b in range(2):
            if out_flight[b] is not None:
                out_flight[b].wait()

    return sc_kernel


def _tc_body(ltau_ref, mpc_ref, dt_ref, idx_ref, out_ref):
    m = mpc_ref[...]
    d = dt_ref[...]
    ii = idx_ref[...]
    lt = jnp.full_like(m, ltau_ref[0])
    for k in range(1, _TBL):
        lt = jnp.where(ii == k, ltau_ref[k], lt)
    h = jnp.exp(d * _DT_SCALE) - 1.0
    decay = jnp.exp(-h * jnp.exp(-lt))
    out_ref[...] = 1.0 - (1.0 - m) * decay


@functools.lru_cache(maxsize=None)
def _build_tc_kernel(n, s):
    rows = (n - s) // _TC_COLS
    row_off = s // _TC_COLS
    blk_off = row_off // _TC_ROWS
    grid = (rows // _TC_ROWS,)
    data_spec = pl.BlockSpec((_TC_ROWS, _TC_COLS), lambda i: (blk_off + i, 0))
    return pl.pallas_call(
        _tc_body,
        grid=grid,
        in_specs=[
            pl.BlockSpec(memory_space=pltpu.SMEM),
            data_spec, data_spec, data_spec,
        ],
        out_specs=pl.BlockSpec((_TC_ROWS, _TC_COLS), lambda i: (i, 0)),
        out_shape=jax.ShapeDtypeStruct((rows, _TC_COLS), jnp.float32),
    )


def kernel(mpc, delta_t, muscle_idx, log_tau):
    n = mpc.shape[0]
    idx = muscle_idx.astype(jnp.int32)
    s = (7 * n // 10) // (_NW * 10240) * (_NW * 10240)
    sc_out = _build_sc_kernel(n, s, 10240, 8)(mpc, delta_t, idx, log_tau)
    mpc2 = mpc.reshape(n // _TC_COLS, _TC_COLS)
    dt2 = delta_t.reshape(n // _TC_COLS, _TC_COLS)
    idx2 = idx.reshape(n // _TC_COLS, _TC_COLS)
    tc_out = _build_tc_kernel(n, s)(log_tau, mpc2, dt2, idx2)
    return lax.dynamic_update_slice(sc_out, tc_out.reshape(-1), (s,))


# speedup vs baseline: 1.9660x; 1.9660x over previous
import functools
import math

import jax
import jax.numpy as jnp
from jax import lax
from jax.experimental import pallas as pl
from jax.experimental.pallas import tpu as pltpu
from jax.experimental.pallas import tpu_sc as plsc

_DT_SCALE = float(math.log1p(168.0))
_NUM_CORES = 2
_NUM_SUBCORES = 16
_NW = _NUM_CORES * _NUM_SUBCORES
_LANES = 16
_TBL = 15

_TC_BLK = 32768


@functools.lru_cache(maxsize=None)
def _build_sc_kernel(n, s, chunk, unroll):
    per_worker = s // _NW
    n_chunks = per_worker // chunk
    assert n_chunks % 2 == 0 and n_chunks >= 4
    mesh = plsc.VectorSubcoreMesh(
        core_axis_name="c", subcore_axis_name="s",
        num_cores=_NUM_CORES, num_subcores=_NUM_SUBCORES)

    @functools.partial(
        pl.kernel,
        out_type=jax.ShapeDtypeStruct((n,), jnp.float32),
        mesh=mesh,
        compiler_params=pltpu.CompilerParams(
            needs_layout_passes=False, skip_device_barrier=True),
        scratch_types=[
            pltpu.VMEM((_LANES,), jnp.float32),
            pltpu.VMEM((_LANES,), jnp.float32),
            [pltpu.VMEM((chunk,), jnp.float32)] * 2,
            [pltpu.VMEM((chunk,), jnp.float32)] * 2,
            [pltpu.VMEM((chunk,), jnp.int32)] * 2,
            [pltpu.VMEM((chunk,), jnp.float32)] * 2,
            [pltpu.SemaphoreType.DMA] * 2,
            [pltpu.SemaphoreType.DMA] * 2,
        ],
    )
    def sc_kernel(mpc_hbm, dt_hbm, idx_hbm, ltau_hbm, out_hbm,
                  ltau_v, tbl_v, mpc_v, dt_v, idx_v, out_v, in_sem, out_sem):
        wid = lax.axis_index("s") * _NUM_CORES + lax.axis_index("c")
        pltpu.sync_copy(ltau_hbm, ltau_v.at[pl.ds(0, _TBL)])
        tbl_v[...] = -jnp.exp(-ltau_v[...])
        base = wid * per_worker

        def in_copies(c, b):
            off = base + c * chunk
            return (
                pltpu.make_async_copy(mpc_hbm.at[pl.ds(off, chunk)], mpc_v[b], in_sem[b]),
                pltpu.make_async_copy(dt_hbm.at[pl.ds(off, chunk)], dt_v[b], in_sem[b]),
                pltpu.make_async_copy(idx_hbm.at[pl.ds(off, chunk)], idx_v[b], in_sem[b]),
            )

        def out_copy(c, b):
            off = base + c * chunk
            return pltpu.make_async_copy(
                out_v[b], out_hbm.at[pl.ds(off, chunk)], out_sem[b])

        def start_in(c, b):
            for d in in_copies(c, b):
                d.start()

        def compute(b):
            @plsc.parallel_loop(0, chunk, step=_LANES, unroll=unroll)
            def _(i):
                sl = pl.ds(pl.multiple_of(i, _LANES), _LANES)
                g = plsc.load_gather(tbl_v, [idx_v[b][sl]])
                e = jnp.exp(dt_v[b][sl] * _DT_SCALE)
                decay = jnp.exp((e - 1.0) * g)
                out_v[b][sl] = 1.0 - (1.0 - mpc_v[b][sl]) * decay

        def step(c, b, p):
            for d in in_copies(c, b):
                d.wait()

            @pl.when(p > 0)
            def _():
                out_copy(c - 2, b).wait()

            compute(b)
            out_copy(c, b).start()

            @pl.when(p + 1 < n_chunks // 2)
            def _():
                start_in(c + 2, b)

        start_in(0, 0)
        start_in(1, 1)

        def pair_body(p, carry):
            c0 = p * 2
            step(c0, 0, p)
            step(c0 + 1, 1, p)
            return carry

        lax.fori_loop(0, n_chunks // 2, pair_body, 0)
        out_copy(n_chunks - 2, 0).wait()
        out_copy(n_chunks - 1, 1).wait()

    return sc_kernel


def _tc_body(nit_ref, mpc_ref, dt_ref, idx_ref, out_ref):
    m = mpc_ref[...]
    d = dt_ref[...]
    ii = idx_ref[...]
    g = jnp.full_like(m, nit_ref[0])
    for k in range(1, _TBL):
        g = jnp.where(ii == k, nit_ref[k], g)
    h = jnp.exp(d * _DT_SCALE) - 1.0
    out_ref[...] = 1.0 - (1.0 - m) * jnp.exp(h * g)


@functools.lru_cache(maxsize=None)
def _build_tc_kernel(n, s):
    m_len = n - s
    blk_off = s // _TC_BLK
    grid = (m_len // _TC_BLK,)
    data_spec = pl.BlockSpec((_TC_BLK,), lambda i: (blk_off + i,))
    return pl.pallas_call(
        _tc_body,
        grid=grid,
        in_specs=[
            pl.BlockSpec(memory_space=pltpu.SMEM),
            data_spec, data_spec, data_spec,
        ],
        out_specs=pl.BlockSpec((_TC_BLK,), lambda i: (i,)),
        out_shape=jax.ShapeDtypeStruct((m_len,), jnp.float32),
    )


def kernel(mpc, delta_t, muscle_idx, log_tau):
    n = mpc.shape[0]
    idx = muscle_idx.astype(jnp.int32)
    chunk = 8960
    s = (7 * n // 10) // (_NW * chunk * 2) * (_NW * chunk * 2)
    sc_out = _build_sc_kernel(n, s, chunk, 8)(mpc, delta_t, idx, log_tau)
    nit = -jnp.exp(-log_tau.astype(jnp.float32))
    tc_out = _build_tc_kernel(n, s)(nit, mpc, delta_t, idx)
    return lax.dynamic_update_slice(sc_out, tc_out, (s,))


# speedup vs baseline: 2.0752x; 1.0555x over previous
import functools
import math

import jax
import jax.numpy as jnp
from jax import lax
from jax.experimental import pallas as pl
from jax.experimental.pallas import tpu as pltpu
from jax.experimental.pallas import tpu_sc as plsc

_DT_SCALE = float(math.log1p(168.0))
_NUM_CORES = 2
_NUM_SUBCORES = 16
_NW = _NUM_CORES * _NUM_SUBCORES
_LANES = 16
_TBL = 15

_TC_BLK = 32768


@functools.lru_cache(maxsize=None)
def _build_sc_kernel(n, s, chunk, unroll):
    per_worker = s // _NW
    n_chunks = per_worker // chunk
    assert n_chunks % 2 == 0 and n_chunks >= 4
    mesh = plsc.VectorSubcoreMesh(
        core_axis_name="c", subcore_axis_name="s",
        num_cores=_NUM_CORES, num_subcores=_NUM_SUBCORES)

    @functools.partial(
        pl.kernel,
        out_type=jax.ShapeDtypeStruct((n,), jnp.float32),
        mesh=mesh,
        compiler_params=pltpu.CompilerParams(
            needs_layout_passes=False, skip_device_barrier=True),
        scratch_types=[
            pltpu.VMEM((_LANES,), jnp.float32),
            pltpu.VMEM((_LANES,), jnp.float32),
            [pltpu.VMEM((chunk,), jnp.float32)] * 2,
            [pltpu.VMEM((chunk,), jnp.float32)] * 2,
            [pltpu.VMEM((chunk,), jnp.int32)] * 2,
            [pltpu.VMEM((chunk,), jnp.float32)] * 2,
            [pltpu.SemaphoreType.DMA] * 2,
            [pltpu.SemaphoreType.DMA] * 2,
        ],
    )
    def sc_kernel(mpc_hbm, dt_hbm, idx_hbm, ltau_hbm, out_hbm,
                  ltau_v, tbl_v, mpc_v, dt_v, idx_v, out_v, in_sem, out_sem):
        wid = lax.axis_index("s") * _NUM_CORES + lax.axis_index("c")
        pltpu.sync_copy(ltau_hbm, ltau_v.at[pl.ds(0, _TBL)])
        tbl_v[...] = -jnp.exp(-ltau_v[...])
        base = wid * per_worker

        def in_copies(c, b):
            off = base + c * chunk
            return (
                pltpu.make_async_copy(mpc_hbm.at[pl.ds(off, chunk)], mpc_v[b], in_sem[b]),
                pltpu.make_async_copy(dt_hbm.at[pl.ds(off, chunk)], dt_v[b], in_sem[b]),
                pltpu.make_async_copy(idx_hbm.at[pl.ds(off, chunk)], idx_v[b], in_sem[b]),
            )

        def out_copy(c, b):
            off = base + c * chunk
            return pltpu.make_async_copy(
                out_v[b], out_hbm.at[pl.ds(off, chunk)], out_sem[b])

        def start_in(c, b):
            for d in in_copies(c, b):
                d.start()

        def compute(b):
            @plsc.parallel_loop(0, chunk, step=_LANES, unroll=unroll)
            def _(i):
                sl = pl.ds(pl.multiple_of(i, _LANES), _LANES)
                g = plsc.load_gather(tbl_v, [idx_v[b][sl]])
                e = jnp.exp(dt_v[b][sl] * _DT_SCALE)
                decay = jnp.exp((e - 1.0) * g)
                out_v[b][sl] = 1.0 - (1.0 - mpc_v[b][sl]) * decay

        def step(c, b, p):
            for d in in_copies(c, b):
                d.wait()

            @pl.when(p > 0)
            def _():
                out_copy(c - 2, b).wait()

            compute(b)
            out_copy(c, b).start()

            @pl.when(p + 1 < n_chunks // 2)
            def _():
                start_in(c + 2, b)

        start_in(0, 0)
        start_in(1, 1)

        def pair_body(p, carry):
            c0 = p * 2
            step(c0, 0, p)
            step(c0 + 1, 1, p)
            return carry

        lax.fori_loop(0, n_chunks // 2, pair_body, 0)
        out_copy(n_chunks - 2, 0).wait()
        out_copy(n_chunks - 1, 1).wait()

    return sc_kernel


def _tc_body(nit_ref, mpc_ref, dt_ref, idx_ref, out_ref):
    m = mpc_ref[...]
    d = dt_ref[...]
    ii = idx_ref[...]
    g = jnp.full_like(m, nit_ref[0])
    for k in range(1, _TBL):
        g = jnp.where(ii == k, nit_ref[k], g)
    h = jnp.exp(d * _DT_SCALE) - 1.0
    out_ref[...] = 1.0 - (1.0 - m) * jnp.exp(h * g)


@functools.lru_cache(maxsize=None)
def _build_tc_kernel(n, s):
    m_len = n - s
    blk_off = s // _TC_BLK
    grid = (m_len // _TC_BLK,)
    data_spec = pl.BlockSpec((_TC_BLK,), lambda i: (blk_off + i,))
    return pl.pallas_call(
        _tc_body,
        grid=grid,
        in_specs=[
            pl.BlockSpec(memory_space=pltpu.SMEM),
            data_spec, data_spec, data_spec,
        ],
        out_specs=pl.BlockSpec((_TC_BLK,), lambda i: (i,)),
        out_shape=jax.ShapeDtypeStruct((m_len,), jnp.float32),
    )


def kernel(mpc, delta_t, muscle_idx, log_tau):
    n = mpc.shape[0]
    idx = muscle_idx.astype(jnp.int32)
    chunk = 8192
    s = (8 * n // 10) // (_NW * chunk * 2) * (_NW * chunk * 2)
    sc_out = _build_sc_kernel(n, s, chunk, 8)(mpc, delta_t, idx, log_tau)
    nit = -jnp.exp(-log_tau.astype(jnp.float32))
    tc_out = _build_tc_kernel(n, s)(nit, mpc, delta_t, idx)
    return lax.dynamic_update_slice(sc_out, tc_out, (s,))


# speedup vs baseline: 2.0928x; 1.0085x over previous
import functools
import math

import jax
import jax.numpy as jnp
from jax import lax
from jax.experimental import pallas as pl
from jax.experimental.pallas import tpu as pltpu
from jax.experimental.pallas import tpu_sc as plsc

_DT_SCALE = float(math.log1p(168.0))
_NUM_CORES = 2
_NUM_SUBCORES = 16
_NW = _NUM_CORES * _NUM_SUBCORES
_LANES = 16
_TBL = 15

_TC_BLK = 32768


@functools.lru_cache(maxsize=None)
def _build_sc_kernel(n, s, chunk, unroll):
    per_worker = s // _NW
    n_chunks = per_worker // chunk
    assert n_chunks % 2 == 0 and n_chunks >= 4
    mesh = plsc.VectorSubcoreMesh(
        core_axis_name="c", subcore_axis_name="s",
        num_cores=_NUM_CORES, num_subcores=_NUM_SUBCORES)

    @functools.partial(
        pl.kernel,
        out_type=jax.ShapeDtypeStruct((n,), jnp.float32),
        mesh=mesh,
        compiler_params=pltpu.CompilerParams(
            needs_layout_passes=False, skip_device_barrier=True),
        scratch_types=[
            pltpu.VMEM((_LANES,), jnp.float32),
            pltpu.VMEM((_LANES,), jnp.float32),
            [pltpu.VMEM((chunk,), jnp.float32)] * 2,
            [pltpu.VMEM((chunk,), jnp.float32)] * 2,
            [pltpu.VMEM((chunk,), jnp.int32)] * 2,
            [pltpu.VMEM((chunk,), jnp.float32)] * 2,
            [pltpu.SemaphoreType.DMA] * 2,
            [pltpu.SemaphoreType.DMA] * 2,
        ],
    )
    def sc_kernel(mpc_hbm, dt_hbm, idx_hbm, ltau_hbm, out_hbm,
                  ltau_v, tbl_v, mpc_v, dt_v, idx_v, out_v, in_sem, out_sem):
        wid = lax.axis_index("s") * _NUM_CORES + lax.axis_index("c")
        pltpu.sync_copy(ltau_hbm, ltau_v.at[pl.ds(0, _TBL)])
        tbl_v[...] = -jnp.exp(-ltau_v[...])
        base = wid * per_worker

        def in_copies(c, b):
            off = base + c * chunk
            return (
                pltpu.make_async_copy(mpc_hbm.at[pl.ds(off, chunk)], mpc_v[b], in_sem[b]),
                pltpu.make_async_copy(dt_hbm.at[pl.ds(off, chunk)], dt_v[b], in_sem[b]),
                pltpu.make_async_copy(idx_hbm.at[pl.ds(off, chunk)], idx_v[b], in_sem[b]),
            )

        def out_copy(c, b):
            off = base + c * chunk
            return pltpu.make_async_copy(
                out_v[b], out_hbm.at[pl.ds(off, chunk)], out_sem[b])

        def start_in(c, b):
            for d in in_copies(c, b):
                d.start()

        def compute(b):
            @plsc.parallel_loop(0, chunk, step=_LANES, unroll=unroll)
            def _(i):
                sl = pl.ds(pl.multiple_of(i, _LANES), _LANES)
                g = plsc.load_gather(tbl_v, [idx_v[b][sl]])
                e = jnp.exp(dt_v[b][sl] * _DT_SCALE)
                decay = jnp.exp((e - 1.0) * g)
                out_v[b][sl] = 1.0 - (1.0 - mpc_v[b][sl]) * decay

        def step(c, b, p):
            for d in in_copies(c, b):
                d.wait()

            @pl.when(p > 0)
            def _():
                out_copy(c - 2, b).wait()

            compute(b)
            out_copy(c, b).start()

            @pl.when(p + 1 < n_chunks // 2)
            def _():
                start_in(c + 2, b)

        start_in(0, 0)
        start_in(1, 1)

        def pair_body(p, carry):
            c0 = p * 2
            step(c0, 0, p)
            step(c0 + 1, 1, p)
            return carry

        lax.fori_loop(0, n_chunks // 2, pair_body, 0)
        out_copy(n_chunks - 2, 0).wait()
        out_copy(n_chunks - 1, 1).wait()

    return sc_kernel


def _tc_body(nit_ref, mpc_ref, dt_ref, idx_ref, out_ref):
    m = mpc_ref[...]
    d = dt_ref[...]
    ii = idx_ref[...]
    g = jnp.full_like(m, nit_ref[0])
    for k in range(1, _TBL):
        g = jnp.where(ii == k, nit_ref[k], g)
    h = jnp.exp(d * _DT_SCALE) - 1.0
    out_ref[...] = 1.0 - (1.0 - m) * jnp.exp(h * g)


@functools.lru_cache(maxsize=None)
def _build_tc_kernel(n, s):
    m_len = n - s
    blk_off = s // _TC_BLK
    grid = (m_len // _TC_BLK,)
    data_spec = pl.BlockSpec((_TC_BLK,), lambda i: (blk_off + i,))
    return pl.pallas_call(
        _tc_body,
        grid=grid,
        in_specs=[
            pl.BlockSpec(memory_space=pltpu.SMEM),
            data_spec, data_spec, data_spec,
        ],
        out_specs=pl.BlockSpec((_TC_BLK,), lambda i: (i,)),
        out_shape=jax.ShapeDtypeStruct((m_len,), jnp.float32),
    )


def kernel(mpc, delta_t, muscle_idx, log_tau):
    n = mpc.shape[0]
    idx = muscle_idx.astype(jnp.int32)
    chunk = 7680
    s = (3 * n // 4) // (_NW * chunk * 2) * (_NW * chunk * 2)
    sc_out = _build_sc_kernel(n, s, chunk, 8)(mpc, delta_t, idx, log_tau)
    nit = -jnp.exp(-log_tau.astype(jnp.float32))
    tc_out = _build_tc_kernel(n, s)(nit, mpc, delta_t, idx)
    return lax.dynamic_update_slice(sc_out, tc_out, (s,))


# speedup vs baseline: 2.1090x; 1.0077x over previous
import functools
import math

import jax
import jax.numpy as jnp
from jax import lax
from jax.experimental import pallas as pl
from jax.experimental.pallas import tpu as pltpu
from jax.experimental.pallas import tpu_sc as plsc

_DT_SCALE = float(math.log1p(168.0))
_NUM_CORES = 2
_NUM_SUBCORES = 16
_NW = _NUM_CORES * _NUM_SUBCORES
_LANES = 16
_TBL = 15

_TC_BLK = 32768


@functools.lru_cache(maxsize=None)
def _build_sc_kernel(n, s, chunk, unroll):
    per_worker = s // _NW
    n_chunks = per_worker // chunk
    assert n_chunks % 2 == 0 and n_chunks >= 4
    mesh = plsc.VectorSubcoreMesh(
        core_axis_name="c", subcore_axis_name="s",
        num_cores=_NUM_CORES, num_subcores=_NUM_SUBCORES)

    @functools.partial(
        pl.kernel,
        out_type=jax.ShapeDtypeStruct((n,), jnp.float32),
        mesh=mesh,
        compiler_params=pltpu.CompilerParams(
            needs_layout_passes=False, skip_device_barrier=True),
        scratch_types=[
            pltpu.VMEM((_LANES,), jnp.float32),
            pltpu.VMEM((_LANES,), jnp.float32),
            [pltpu.VMEM((chunk,), jnp.float32)] * 2,
            [pltpu.VMEM((chunk,), jnp.float32)] * 2,
            [pltpu.VMEM((chunk,), jnp.int32)] * 2,
            [pltpu.VMEM((chunk,), jnp.float32)] * 2,
            [pltpu.SemaphoreType.DMA] * 2,
            [pltpu.SemaphoreType.DMA] * 2,
        ],
    )
    def sc_kernel(mpc_hbm, dt_hbm, idx_hbm, ltau_hbm, out_hbm,
                  ltau_v, tbl_v, mpc_v, dt_v, idx_v, out_v, in_sem, out_sem):
        wid = lax.axis_index("s") * _NUM_CORES + lax.axis_index("c")
        pltpu.sync_copy(ltau_hbm, ltau_v.at[pl.ds(0, _TBL)])
        tbl_v[...] = -jnp.exp(-ltau_v[...])
        base = wid * per_worker

        def in_copies(c, b):
            off = base + c * chunk
            return (
                pltpu.make_async_copy(mpc_hbm.at[pl.ds(off, chunk)], mpc_v[b], in_sem[b]),
                pltpu.make_async_copy(dt_hbm.at[pl.ds(off, chunk)], dt_v[b], in_sem[b]),
                pltpu.make_async_copy(idx_hbm.at[pl.ds(off, chunk)], idx_v[b], in_sem[b]),
            )

        def out_copy(c, b):
            off = base + c * chunk
            return pltpu.make_async_copy(
                out_v[b], out_hbm.at[pl.ds(off, chunk)], out_sem[b])

        def start_in(c, b):
            for d in in_copies(c, b):
                d.start()

        def compute(b):
            @plsc.parallel_loop(0, chunk, step=_LANES, unroll=unroll)
            def _(i):
                sl = pl.ds(pl.multiple_of(i, _LANES), _LANES)
                g = plsc.load_gather(tbl_v, [idx_v[b][sl]])
                e = jnp.exp(dt_v[b][sl] * _DT_SCALE)
                decay = jnp.exp((e - 1.0) * g)
                out_v[b][sl] = 1.0 - (1.0 - mpc_v[b][sl]) * decay

        def step(c, b, p):
            for d in in_copies(c, b):
                d.wait()

            @pl.when(p > 0)
            def _():
                out_copy(c - 2, b).wait()

            compute(b)
            out_copy(c, b).start()

            @pl.when(p + 1 < n_chunks // 2)
            def _():
                start_in(c + 2, b)

        start_in(0, 0)
        start_in(1, 1)

        def pair_body(p, carry):
            c0 = p * 2
            step(c0, 0, p)
            step(c0 + 1, 1, p)
            return carry

        lax.fori_loop(0, n_chunks // 2, pair_body, 0)
        out_copy(n_chunks - 2, 0).wait()
        out_copy(n_chunks - 1, 1).wait()

    return sc_kernel


def _tc_body(nit_ref, mpc_ref, dt_ref, idx_ref, out_ref):
    m = mpc_ref[...]
    d = dt_ref[...]
    ii = idx_ref[...]
    g = jnp.full_like(m, nit_ref[0])
    for k in range(1, _TBL):
        g = jnp.where(ii == k, nit_ref[k], g)
    h = jnp.exp(d * _DT_SCALE) - 1.0
    out_ref[...] = 1.0 - (1.0 - m) * jnp.exp(h * g)


@functools.lru_cache(maxsize=None)
def _build_tc_kernel(n, s):
    m_len = n - s
    blk_off = s // _TC_BLK
    grid = (m_len // _TC_BLK,)
    data_spec = pl.BlockSpec((_TC_BLK,), lambda i: (blk_off + i,))
    return pl.pallas_call(
        _tc_body,
        grid=grid,
        in_specs=[
            pl.BlockSpec(memory_space=pltpu.SMEM),
            data_spec, data_spec, data_spec,
        ],
        out_specs=pl.BlockSpec((_TC_BLK,), lambda i: (i,)),
        out_shape=jax.ShapeDtypeStruct((m_len,), jnp.float32),
    )


def kernel(mpc, delta_t, muscle_idx, log_tau):
    n = mpc.shape[0]
    idx = muscle_idx.astype(jnp.int32)
    chunk = 7680
    s = (3 * n // 4) // (_NW * chunk * 2) * (_NW * chunk * 2)
    sc_out = _build_sc_kernel(n, s, chunk, 4)(mpc, delta_t, idx, log_tau)
    nit = -jnp.exp(-log_tau.astype(jnp.float32))
    tc_out = _build_tc_kernel(n, s)(nit, mpc, delta_t, idx)
    return lax.dynamic_update_slice(sc_out, tc_out, (s,))
